# baseline (speedup 1.0000x reference)
import functools

import jax
import jax.numpy as jnp
from jax import lax
from jax.experimental import pallas as pl
from jax.experimental.pallas import tpu as pltpu, tpu_sc as plsc

NUM_HEADS = 8
HEAD_DIM = 64
STORAGE_SIZE = 1024
CHUNK_SIZE = 64
RETRIEVAL_K = 8
SEQ_LEN = 512
BATCH = 4

def _fused_kernel(k_ref, v_ref, q_ref, ckv_ref, idx_ref, rv_ref, ri_ref,
                  *, C, Dh, NB, NBLK, B, S, K, N):
    h = pl.program_id(0)
    nb = pl.program_id(1)

    def seq_mean(x):
        acc = x[:, 0:Dh] + x[:, Dh : 2 * Dh]
        for cp in range(1, C // 2):
            off = cp * 2 * Dh
            acc = (acc + x[:, off : off + Dh]) + x[:, off + Dh : off + 2 * Dh]
        return acc * (1.0 / C)

    mk = seq_mean(k_ref[0])
    mv = seq_mean(v_ref[0])
    ckv_ref[0] = jnp.concatenate([mk, mv], axis=-1)

    iota_g = jax.lax.broadcasted_iota(jnp.int32, (S, NB), 1) + nb * NB
    for b in range(B):
        q = q_ref[0, b]
        sv = jax.lax.dot_general(
            q, mk, (((1,), (1,)), ((), ())), preferred_element_type=jnp.float32
        )
        vcols, icols = [], []
        for k in range(K):
            m = jnp.max(sv, axis=1, keepdims=True)
            arg = jnp.min(jnp.where(sv == m, iota_g, (1 << 30)), axis=1, keepdims=True)
            vcols.append(m)
            icols.append(arg)
            if k < K - 1:
                sv = jnp.where(iota_g == arg, -jnp.inf, sv)
        slab_v = jnp.concatenate(vcols, axis=1)
        slab_i = jnp.concatenate(icols, axis=1)

        prev_v = jnp.where(nb == 0, -jnp.inf, rv_ref[b])
        prev_i = jnp.where(nb == 0, (1 << 30), ri_ref[b])
        cand_v = jnp.concatenate([prev_v, slab_v], axis=1)
        cand_i = jnp.concatenate([prev_i, slab_i], axis=1)
        mvcols, micols = [], []
        for k in range(K):
            m = jnp.max(cand_v, axis=1, keepdims=True)
            pick = jnp.min(
                jnp.where(cand_v == m, cand_i, (1 << 30)), axis=1, keepdims=True
            )
            mvcols.append(m)
            micols.append(pick)
            if k < K - 1:
                kill = jnp.logical_and(cand_v == m, cand_i == pick)
                cand_v = jnp.where(kill, -jnp.inf, cand_v)
        rv_ref[b] = jnp.concatenate(mvcols, axis=1)
        ri_ref[b] = jnp.concatenate(micols, axis=1)

        @pl.when(nb == NBLK - 1)
        def _emit():
            idx_ref[0, b] = ri_ref[b] + h * N


def _make_sc_gather(R, W, n_workers, chunk):
    rows_per_w = R // n_workers
    n_chunks = rows_per_w // chunk
    mesh = plsc.VectorSubcoreMesh(core_axis_name="c", subcore_axis_name="s")

    @functools.partial(
        pl.kernel,
        mesh=mesh,
        out_type=jax.ShapeDtypeStruct((R, W), jnp.float32),
        scratch_types=[
            pltpu.VMEM((chunk,), jnp.int32),
            pltpu.VMEM((chunk, W), jnp.float32),
            pltpu.SemaphoreType.DMA,
        ],
    )
    def gather_k(table_hbm, idx_hbm, out_hbm, idx_v, rows_v, sem):
        wid = lax.axis_index("s") * 2 + lax.axis_index("c")
        base = wid * rows_per_w

        @pl.loop(0, n_chunks)
        def _chunk(j):
            off = base + j * chunk
            pltpu.sync_copy(idx_hbm.at[pl.ds(off, chunk)], idx_v)
            pltpu.async_copy(table_hbm.at[idx_v], rows_v, sem).wait()
            pltpu.sync_copy(rows_v, out_hbm.at[pl.ds(off, chunk)])

    return gather_k


def kernel(queries, keys, vals):
    S, B, D = queries.shape
    H, N, C, Dh = keys.shape
    K = RETRIEVAL_K
    W = 2 * Dh

    NB = 128
    NBLK = N // NB
    keys_l = keys.reshape(H, N, C * Dh)
    vals_l = vals.reshape(H, N, C * Dh)
    qh = queries.reshape(S, B, H, Dh).transpose(2, 1, 0, 3)

    ckv, top_idx = pl.pallas_call(
        functools.partial(
            _fused_kernel, C=C, Dh=Dh, NB=NB, NBLK=NBLK, B=B, S=S, K=K, N=N
        ),
        grid=(H, NBLK),
        in_specs=[
            pl.BlockSpec((1, NB, C * Dh), lambda h, n: (h, n, 0)),
            pl.BlockSpec((1, NB, C * Dh), lambda h, n: (h, n, 0)),
            pl.BlockSpec((1, B, S, Dh), lambda h, n: (h, 0, 0, 0)),
        ],
        out_specs=[
            pl.BlockSpec((1, NB, W), lambda h, n: (h, n, 0)),
            pl.BlockSpec((1, B, S, K), lambda h, n: (h, 0, 0, 0)),
        ],
        out_shape=[
            jax.ShapeDtypeStruct((H, N, W), jnp.float32),
            jax.ShapeDtypeStruct((H, B, S, K), jnp.int32),
        ],
        scratch_shapes=[
            pltpu.VMEM((B, S, K), jnp.float32),
            pltpu.VMEM((B, S, K), jnp.int32),
        ],
    )(keys_l, vals_l, qh)

    R = B * H * S * K
    idx_flat = top_idx.transpose(1, 0, 2, 3).reshape(R)
    gather = _make_sc_gather(R, W, n_workers=32, chunk=128)
    comb = gather(ckv.reshape(H * N, W), idx_flat)
    comb = comb.reshape(B * H, S, K, W)
    return (comb[..., :Dh], comb[..., Dh:])


# speedup vs baseline: 2.6498x; 2.6498x over previous
import functools

import jax
import jax.numpy as jnp
from jax import lax
from jax.experimental import pallas as pl
from jax.experimental.pallas import tpu as pltpu, tpu_sc as plsc

NUM_HEADS = 8
HEAD_DIM = 64
STORAGE_SIZE = 1024
CHUNK_SIZE = 64
RETRIEVAL_K = 8
SEQ_LEN = 512
BATCH = 4


def _mean_kernel(k_ref, v_ref, ckv_ref, *, C, Dh):
    def seq_mean(x):
        acc = x[:, 0:Dh] + x[:, Dh : 2 * Dh]
        for cp in range(1, C // 2):
            off = cp * 2 * Dh
            acc = (acc + x[:, off : off + Dh]) + x[:, off + Dh : off + 2 * Dh]
        return acc * (1.0 / C)

    ckv_ref[0] = jnp.concatenate(
        [seq_mean(k_ref[0]), seq_mean(v_ref[0])], axis=-1
    )


def _topk_kernel(q_ref, ckv_ref, idx_ref, *, S, N, K, Dh):
    h = pl.program_id(1)
    q = q_ref[0, 0]
    ck = ckv_ref[0][:, :Dh]
    scores = jax.lax.dot_general(
        q, ck, (((1,), (1,)), ((), ())), preferred_element_type=jnp.float32
    )
    iota = jax.lax.broadcasted_iota(jnp.int32, (S, N), 1)
    cols = []
    for k in range(K):
        m = jnp.max(scores, axis=1, keepdims=True)
        sel = jnp.where(scores == m, iota, N)
        arg = jnp.min(sel, axis=1, keepdims=True)
        cols.append(arg)
        if k < K - 1:
            scores = jnp.where(iota == arg, -jnp.inf, scores)
    idx = jnp.concatenate(cols, axis=1)
    idx_ref[0] = idx + h * N


def _make_sc_gather(R, W, n_workers, chunk):
    rows_per_w = R // n_workers
    n_chunks = rows_per_w // chunk
    mesh = plsc.VectorSubcoreMesh(core_axis_name="c", subcore_axis_name="s")

    @functools.partial(
        pl.kernel,
        mesh=mesh,
        out_type=jax.ShapeDtypeStruct((R, W), jnp.float32),
        scratch_types=[
            pltpu.VMEM((chunk,), jnp.int32),
            pltpu.VMEM((chunk,), jnp.int32),
            pltpu.VMEM((chunk, W), jnp.float32),
            pltpu.VMEM((chunk, W), jnp.float32),
            pltpu.SemaphoreType.DMA,
            pltpu.SemaphoreType.DMA,
        ],
    )
    def gather_k(table_hbm, idx_hbm, out_hbm, idx_v0, idx_v1, rows_v0, rows_v1, sem0, sem1):
        wid = lax.axis_index("s") * 2 + lax.axis_index("c")
        base = wid * rows_per_w

        pltpu.sync_copy(idx_hbm.at[pl.ds(base, chunk)], idx_v0)
        pltpu.async_copy(table_hbm.at[idx_v0], rows_v0, sem0)

        @pl.loop(0, n_chunks // 2)
        def _chunk(jj):
            off0 = base + (2 * jj) * chunk
            off1 = off0 + chunk
            pltpu.sync_copy(idx_hbm.at[pl.ds(off1, chunk)], idx_v1)
            pltpu.async_copy(table_hbm.at[idx_v1], rows_v1, sem1)
            pltpu.make_async_copy(table_hbm.at[idx_v0], rows_v0, sem0).wait()
            pltpu.sync_copy(rows_v0, out_hbm.at[pl.ds(off0, chunk)])

            @pl.when(jj < n_chunks // 2 - 1)
            def _next():
                off2 = off1 + chunk
                pltpu.sync_copy(idx_hbm.at[pl.ds(off2, chunk)], idx_v0)
                pltpu.async_copy(table_hbm.at[idx_v0], rows_v0, sem0)

            pltpu.make_async_copy(table_hbm.at[idx_v1], rows_v1, sem1).wait()
            pltpu.sync_copy(rows_v1, out_hbm.at[pl.ds(off1, chunk)])

    return gather_k


def kernel(queries, keys, vals):
    S, B, D = queries.shape
    H, N, C, Dh = keys.shape
    K = RETRIEVAL_K
    W = 2 * Dh

    NB = 128
    keys_l = keys.reshape(H, N, C * Dh)
    vals_l = vals.reshape(H, N, C * Dh)
    ckv = pl.pallas_call(
        functools.partial(_mean_kernel, C=C, Dh=Dh),
        grid=(H, N // NB),
        in_specs=[
            pl.BlockSpec((1, NB, C * Dh), lambda h, n: (h, n, 0)),
            pl.BlockSpec((1, NB, C * Dh), lambda h, n: (h, n, 0)),
        ],
        out_specs=pl.BlockSpec((1, NB, W), lambda h, n: (h, n, 0)),
        out_shape=jax.ShapeDtypeStruct((H, N, W), jnp.float32),
    )(keys_l, vals_l)

    q4 = queries.reshape(S, B, H, Dh).transpose(1, 2, 0, 3)

    top_idx = pl.pallas_call(
        functools.partial(_topk_kernel, S=S, N=N, K=K, Dh=Dh),
        grid=(B, H),
        in_specs=[
            pl.BlockSpec((1, 1, S, Dh), lambda b, h: (b, h, 0, 0)),
            pl.BlockSpec((1, N, W), lambda b, h: (h, 0, 0)),
        ],
        out_specs=pl.BlockSpec((1, S, K), lambda b, h: (b * NUM_HEADS + h, 0, 0)),
        out_shape=jax.ShapeDtypeStruct((B * H, S, K), jnp.int32),
    )(q4, ckv)

    R = B * H * S * K
    gather = _make_sc_gather(R, W, n_workers=32, chunk=128)
    comb = gather(ckv.reshape(H * N, W), top_idx.reshape(R))
    comb = comb.reshape(B * H, S, K, W)
    return (comb[..., :Dh], comb[..., Dh:])


# speedup vs baseline: 2.7025x; 1.0199x over previous
import functools

import jax
import jax.numpy as jnp
from jax import lax
from jax.experimental import pallas as pl
from jax.experimental.pallas import tpu as pltpu, tpu_sc as plsc

NUM_HEADS = 8
HEAD_DIM = 64
STORAGE_SIZE = 1024
CHUNK_SIZE = 64
RETRIEVAL_K = 8
SEQ_LEN = 512
BATCH = 4


def _mean_kernel(k_ref, v_ref, ckv_ref, *, C, Dh):
    def seq_mean(x):
        acc = x[:, 0:Dh] + x[:, Dh : 2 * Dh]
        for cp in range(1, C // 2):
            off = cp * 2 * Dh
            acc = (acc + x[:, off : off + Dh]) + x[:, off + Dh : off + 2 * Dh]
        return acc * (1.0 / C)

    ckv_ref[0] = jnp.concatenate(
        [seq_mean(k_ref[0]), seq_mean(v_ref[0])], axis=-1
    )


def _topk_kernel(q_ref, ckv_ref, idx_ref, *, S, N, K, Dh):
    h = pl.program_id(1)
    q = q_ref[0, 0]
    ck = ckv_ref[0][:, :Dh]
    scores = jax.lax.dot_general(
        q, ck, (((1,), (1,)), ((), ())), preferred_element_type=jnp.float32
    )
    iota = jax.lax.broadcasted_iota(jnp.int32, (S, N), 1)
    cols = []
    for k in range(K):
        m = jnp.max(scores, axis=1, keepdims=True)
        sel = jnp.where(scores == m, iota, N)
        arg = jnp.min(sel, axis=1, keepdims=True)
        cols.append(arg)
        if k < K - 1:
            scores = jnp.where(iota == arg, -jnp.inf, scores)
    idx = jnp.concatenate(cols, axis=1)
    idx_ref[0] = idx + h * N


def _make_sc_gather(R, W, n_workers, chunk):
    rows_per_w = R // n_workers
    n_chunks = rows_per_w // chunk
    mesh = plsc.VectorSubcoreMesh(core_axis_name="c", subcore_axis_name="s")

    @functools.partial(
        pl.kernel,
        mesh=mesh,
        out_type=jax.ShapeDtypeStruct((R, W), jnp.float32),
        scratch_types=[
            pltpu.VMEM((chunk,), jnp.int32),
            pltpu.VMEM((chunk,), jnp.int32),
            pltpu.VMEM((chunk, W), jnp.float32),
            pltpu.VMEM((chunk, W), jnp.float32),
            pltpu.SemaphoreType.DMA,
            pltpu.SemaphoreType.DMA,
        ],
    )
    def gather_k(table_hbm, idx_hbm, out_hbm, idx_v0, idx_v1, rows_v0, rows_v1, sem0, sem1):
        wid = lax.axis_index("s") * 2 + lax.axis_index("c")
        base = wid * rows_per_w

        pltpu.sync_copy(idx_hbm.at[pl.ds(base, chunk)], idx_v0)
        pltpu.async_copy(table_hbm.at[idx_v0], rows_v0, sem0)

        @pl.loop(0, n_chunks // 2)
        def _chunk(jj):
            off0 = base + (2 * jj) * chunk
            off1 = off0 + chunk
            pltpu.sync_copy(idx_hbm.at[pl.ds(off1, chunk)], idx_v1)
            pltpu.async_copy(table_hbm.at[idx_v1], rows_v1, sem1)
            pltpu.make_async_copy(table_hbm.at[idx_v0], rows_v0, sem0).wait()
            pltpu.sync_copy(rows_v0, out_hbm.at[pl.ds(off0, chunk)])

            @pl.when(jj < n_chunks // 2 - 1)
            def _next():
                off2 = off1 + chunk
                pltpu.sync_copy(idx_hbm.at[pl.ds(off2, chunk)], idx_v0)
                pltpu.async_copy(table_hbm.at[idx_v0], rows_v0, sem0)

            pltpu.make_async_copy(table_hbm.at[idx_v1], rows_v1, sem1).wait()
            pltpu.sync_copy(rows_v1, out_hbm.at[pl.ds(off1, chunk)])

    return gather_k


def kernel(queries, keys, vals):
    S, B, D = queries.shape
    H, N, C, Dh = keys.shape
    K = RETRIEVAL_K
    W = 2 * Dh

    NB = 256
    keys_l = keys.reshape(H, N, C * Dh)
    vals_l = vals.reshape(H, N, C * Dh)
    ckv = pl.pallas_call(
        functools.partial(_mean_kernel, C=C, Dh=Dh),
        grid=(H, N // NB),
        in_specs=[
            pl.BlockSpec((1, NB, C * Dh), lambda h, n: (h, n, 0)),
            pl.BlockSpec((1, NB, C * Dh), lambda h, n: (h, n, 0)),
        ],
        out_specs=pl.BlockSpec((1, NB, W), lambda h, n: (h, n, 0)),
        out_shape=jax.ShapeDtypeStruct((H, N, W), jnp.float32),
    )(keys_l, vals_l)

    q4 = queries.reshape(S, B, H, Dh).transpose(1, 2, 0, 3)

    top_idx = pl.pallas_call(
        functools.partial(_topk_kernel, S=S, N=N, K=K, Dh=Dh),
        grid=(B, H),
        in_specs=[
            pl.BlockSpec((1, 1, S, Dh), lambda b, h: (b, h, 0, 0)),
            pl.BlockSpec((1, N, W), lambda b, h: (h, 0, 0)),
        ],
        out_specs=pl.BlockSpec((1, S, K), lambda b, h: (b * NUM_HEADS + h, 0, 0)),
        out_shape=jax.ShapeDtypeStruct((B * H, S, K), jnp.int32),
    )(q4, ckv)

    R = B * H * S * K
    gather = _make_sc_gather(R, W, n_workers=32, chunk=128)
    comb = gather(ckv.reshape(H * N, W), top_idx.reshape(R))
    comb = comb.reshape(B * H, S, K, W)
    return (comb[..., :Dh], comb[..., Dh:])


# speedup vs baseline: 2.7627x; 1.0223x over previous
import functools

import jax
import jax.numpy as jnp
from jax import lax
from jax.experimental import pallas as pl
from jax.experimental.pallas import tpu as pltpu, tpu_sc as plsc

NUM_HEADS = 8
HEAD_DIM = 64
STORAGE_SIZE = 1024
CHUNK_SIZE = 64
RETRIEVAL_K = 8
SEQ_LEN = 512
BATCH = 4


def _mean_kernel(k_ref, v_ref, ckv_ref, *, C, Dh):
    def seq_mean(x):
        acc = x[:, 0:Dh] + x[:, Dh : 2 * Dh]
        for cp in range(1, C // 2):
            off = cp * 2 * Dh
            acc = (acc + x[:, off : off + Dh]) + x[:, off + Dh : off + 2 * Dh]
        return acc * (1.0 / C)

    ckv_ref[0] = jnp.concatenate(
        [seq_mean(k_ref[0]), seq_mean(v_ref[0])], axis=-1
    )


def _topk_kernel(q_ref, ckv_ref, idx_ref, *, S, N, K, Dh):
    h = pl.program_id(1)
    q = q_ref[0, 0]
    ck = ckv_ref[0][:, :Dh]
    scores = jax.lax.dot_general(
        q, ck, (((1,), (1,)), ((), ())), preferred_element_type=jnp.float32
    )
    iota = jax.lax.broadcasted_iota(jnp.int32, (S, N), 1)
    cols = []
    for k in range(K):
        m = jnp.max(scores, axis=1, keepdims=True)
        sel = jnp.where(scores == m, iota, N)
        arg = jnp.min(sel, axis=1, keepdims=True)
        cols.append(arg)
        if k < K - 1:
            scores = jnp.where(iota == arg, -jnp.inf, scores)
    idx = jnp.concatenate(cols, axis=1)
    idx_ref[0] = idx + h * N


def _make_sc_gather(R, W, n_workers, chunk):
    rows_per_w = R // n_workers
    n_chunks = rows_per_w // chunk
    mesh = plsc.VectorSubcoreMesh(core_axis_name="c", subcore_axis_name="s")

    @functools.partial(
        pl.kernel,
        mesh=mesh,
        out_type=jax.ShapeDtypeStruct((R, W), jnp.float32),
        scratch_types=[
            pltpu.VMEM((chunk,), jnp.int32),
            pltpu.VMEM((chunk,), jnp.int32),
            pltpu.VMEM((chunk, W), jnp.float32),
            pltpu.VMEM((chunk, W), jnp.float32),
            pltpu.SemaphoreType.DMA,
            pltpu.SemaphoreType.DMA,
        ],
    )
    def gather_k(table_hbm, idx_hbm, out_hbm, idx_v0, idx_v1, rows_v0, rows_v1, sem0, sem1):
        wid = lax.axis_index("s") * 2 + lax.axis_index("c")
        base = wid * rows_per_w

        pltpu.sync_copy(idx_hbm.at[pl.ds(base, chunk)], idx_v0)
        pltpu.async_copy(table_hbm.at[idx_v0], rows_v0, sem0)

        @pl.loop(0, n_chunks // 2)
        def _chunk(jj):
            off0 = base + (2 * jj) * chunk
            off1 = off0 + chunk
            pltpu.sync_copy(idx_hbm.at[pl.ds(off1, chunk)], idx_v1)
            pltpu.async_copy(table_hbm.at[idx_v1], rows_v1, sem1)
            pltpu.make_async_copy(table_hbm.at[idx_v0], rows_v0, sem0).wait()
            pltpu.sync_copy(rows_v0, out_hbm.at[pl.ds(off0, chunk)])

            @pl.when(jj < n_chunks // 2 - 1)
            def _next():
                off2 = off1 + chunk
                pltpu.sync_copy(idx_hbm.at[pl.ds(off2, chunk)], idx_v0)
                pltpu.async_copy(table_hbm.at[idx_v0], rows_v0, sem0)

            pltpu.make_async_copy(table_hbm.at[idx_v1], rows_v1, sem1).wait()
            pltpu.sync_copy(rows_v1, out_hbm.at[pl.ds(off1, chunk)])

    return gather_k


def kernel(queries, keys, vals):
    S, B, D = queries.shape
    H, N, C, Dh = keys.shape
    K = RETRIEVAL_K
    W = 2 * Dh

    NB = 512
    keys_l = keys.reshape(H, N, C * Dh)
    vals_l = vals.reshape(H, N, C * Dh)
    ckv = pl.pallas_call(
        functools.partial(_mean_kernel, C=C, Dh=Dh),
        grid=(H, N // NB),
        in_specs=[
            pl.BlockSpec((1, NB, C * Dh), lambda h, n: (h, n, 0)),
            pl.BlockSpec((1, NB, C * Dh), lambda h, n: (h, n, 0)),
        ],
        out_specs=pl.BlockSpec((1, NB, W), lambda h, n: (h, n, 0)),
        out_shape=jax.ShapeDtypeStruct((H, N, W), jnp.float32),
    )(keys_l, vals_l)

    q4 = queries.reshape(S, B, H, Dh).transpose(1, 2, 0, 3)

    top_idx = pl.pallas_call(
        functools.partial(_topk_kernel, S=S, N=N, K=K, Dh=Dh),
        grid=(B, H),
        in_specs=[
            pl.BlockSpec((1, 1, S, Dh), lambda b, h: (b, h, 0, 0)),
            pl.BlockSpec((1, N, W), lambda b, h: (h, 0, 0)),
        ],
        out_specs=pl.BlockSpec((1, S, K), lambda b, h: (b * NUM_HEADS + h, 0, 0)),
        out_shape=jax.ShapeDtypeStruct((B * H, S, K), jnp.int32),
    )(q4, ckv)

    R = B * H * S * K
    gather = _make_sc_gather(R, W, n_workers=32, chunk=128)
    comb = gather(ckv.reshape(H * N, W), top_idx.reshape(R))
    comb = comb.reshape(B * H, S, K, W)
    return (comb[..., :Dh], comb[..., Dh:])
